# Initial kernel scaffold; baseline (speedup 1.0000x reference)
import jax
import jax.numpy as jnp
from jax.experimental import pallas as pl


def kernel(x, Wg, W1, b1, W2, b2):
    raise NotImplementedError("write your pallas kernel here")


# speedup vs baseline: 1.0405x; 1.0405x over previous
import functools
import math

import jax
import jax.numpy as jnp
from jax.experimental import pallas as pl
from jax.experimental.pallas import tpu as pltpu

_TM = 256


def _gelu(x):
    return 0.5 * x * (1.0 + jnp.tanh(math.sqrt(2.0 / math.pi) * (x + 0.044715 * x ** 3)))


def _moe_body(x_ref, wg_ref, w1_ref, b1_ref, w2_ref, b2_ref, out_ref, loss_ref,
              *, n_tokens, n_experts):
    j = pl.program_id(0)
    nsteps = pl.num_programs(0)

    xt = x_ref[...]
    g = jnp.dot(xt, wg_ref[...], preferred_element_type=jnp.float32)

    gsum = jnp.sum(g)

    @pl.when(j == 0)
    def _init():
        loss_ref[0, 0] = 0.0

    loss_ref[0, 0] += gsum

    @pl.when(j == nsteps - 1)
    def _fini():
        m = loss_ref[0, 0] / (n_tokens * n_experts)
        loss_ref[0, 0] = m * jnp.log(m + 0.1)

    ids = jax.lax.broadcasted_iota(jnp.int32, g.shape, 1)
    w0 = jnp.max(g, axis=1, keepdims=True)
    e0 = jnp.min(jnp.where(g == w0, ids, n_experts), axis=1, keepdims=True)
    first = ids == e0
    g2 = jnp.where(first, -jnp.inf, g)
    w1 = jnp.max(g2, axis=1, keepdims=True)
    e1 = jnp.min(jnp.where(g2 == w1, ids, n_experts), axis=1, keepdims=True)
    second = ids == e1
    z = jnp.exp(w1 - w0)
    denom = 1.0 + z
    coeff = (first.astype(jnp.float32) + second.astype(jnp.float32) * z) / denom

    acc = jnp.zeros_like(xt)
    for i in range(n_experts):
        h = _gelu(jnp.dot(xt, w1_ref[i], preferred_element_type=jnp.float32)
                  + b1_ref[i][None, :])
        y = jnp.dot(h, w2_ref[i], preferred_element_type=jnp.float32) + b2_ref[i][None, :]
        acc = acc + coeff[:, i:i + 1] * y
    out_ref[...] = acc


def kernel(x, Wg, W1, b1, W2, b2):
    orig_shape = x.shape
    d = x.shape[-1]
    flat = x.reshape(-1, d)
    n_tokens = flat.shape[0]
    n_experts = W1.shape[0]
    d_ff = W1.shape[2]
    grid = (n_tokens // _TM,)

    body = functools.partial(_moe_body, n_tokens=n_tokens, n_experts=n_experts)
    out, loss = pl.pallas_call(
        body,
        grid=grid,
        in_specs=[
            pl.BlockSpec((_TM, d), lambda j: (j, 0)),
            pl.BlockSpec((d, n_experts), lambda j: (0, 0)),
            pl.BlockSpec((n_experts, d, d_ff), lambda j: (0, 0, 0)),
            pl.BlockSpec((n_experts, d_ff), lambda j: (0, 0)),
            pl.BlockSpec((n_experts, d_ff, d), lambda j: (0, 0, 0)),
            pl.BlockSpec((n_experts, d), lambda j: (0, 0)),
        ],
        out_specs=[
            pl.BlockSpec((_TM, d), lambda j: (j, 0)),
            pl.BlockSpec((1, 1), lambda j: (0, 0), memory_space=pltpu.SMEM),
        ],
        out_shape=[
            jax.ShapeDtypeStruct((n_tokens, d), jnp.float32),
            jax.ShapeDtypeStruct((1, 1), jnp.float32),
        ],
    )(flat, Wg, W1, b1, W2, b2)
    return out.reshape(orig_shape), loss.reshape(())
